# baseline (speedup 1.0000x reference)
import functools

import jax
import jax.numpy as jnp
from jax import lax
from jax.experimental import pallas as pl
from jax.experimental.pallas import tpu as pltpu
from jax.experimental.pallas import tpu_sc as plsc

N_T = 26
VOCAB = 100000
D = 32
BATCH = 4096
L = 20

BAGS = N_T * BATCH
NW = 32
BAGS_PER_W = BAGS // NW
G = 64
CHUNKS = BAGS_PER_W // G
IDX_PER_CHUNK = G * L
GW = 128
K = IDX_PER_CHUNK // GW
IDX_ROWS_PER_W = BAGS_PER_W * L // GW


def _pooled_sc(idx3d, flat_tab):
    mesh = plsc.VectorSubcoreMesh(core_axis_name="c", subcore_axis_name="s")

    @functools.partial(
        pl.kernel,
        out_type=jax.ShapeDtypeStruct((BAGS, D), jnp.float32),
        mesh=mesh,
        scratch_types=[
            pltpu.VMEM((K, GW), jnp.int32),
            pltpu.VMEM((IDX_PER_CHUNK, D), jnp.float32),
            pltpu.VMEM((G, D), jnp.float32),
            pltpu.SemaphoreType.DMA,
        ],
        compiler_params=pltpu.CompilerParams(use_tc_tiling_on_sc=False),
    )
    def k(idx_hbm, tab_hbm, out_hbm, idx_v, rows_v, out_v, sem):
        wid = lax.axis_index("s") * 2 + lax.axis_index("c")
        bag_base = wid * BAGS_PER_W

        @pl.loop(0, CHUNKS)
        def _(c):
            bag0 = bag_base + c * G
            pltpu.sync_copy(idx_hbm.at[wid * CHUNKS + c], idx_v)
            copies = []
            for j in range(K):
                copies.append(
                    pltpu.async_copy(
                        tab_hbm.at[idx_v.at[j]],
                        rows_v.at[pl.ds(j * GW, GW)],
                        sem,
                    )
                )
            for cp in copies:
                cp.wait()

            @pl.loop(0, G)
            def _(g):
                r0 = g * L
                a0 = rows_v[r0, pl.ds(0, 16)]
                a1 = rows_v[r0, pl.ds(16, 16)]
                for step in range(1, L):
                    a0 = a0 + rows_v[r0 + step, pl.ds(0, 16)]
                    a1 = a1 + rows_v[r0 + step, pl.ds(16, 16)]
                out_v[g, pl.ds(0, 16)] = a0
                out_v[g, pl.ds(16, 16)] = a1

            pltpu.sync_copy(out_v, out_hbm.at[pl.ds(bag0, G)])

    return k(idx3d, flat_tab)


VB = 3200
NJ = -(-VOCAB // VB)


def _detile_tc(tab_t):
    def body(x_ref, o_ref):
        x = x_ref[0]
        y = jnp.transpose(x.reshape(D, VB // 4, 4), (1, 2, 0))
        o_ref[0] = y.reshape(VB // 4, 4 * D)

    return pl.pallas_call(
        body,
        grid=(N_T, NJ),
        in_specs=[pl.BlockSpec((1, D, VB), lambda t, j: (t, 0, j))],
        out_specs=pl.BlockSpec((1, VB // 4, 4 * D), lambda t, j: (t, j, 0)),
        out_shape=jax.ShapeDtypeStruct((N_T, VOCAB // 4, 4 * D), jnp.float32),
    )(tab_t)


BLK = 2048
PR = BAGS // 4


def _mlp_tc(x128, w1, c1, w2, c2, w3, c3):
    def body(x_ref, w1_ref, c1_ref, w2_ref, c2_ref, w3_ref, c3_ref, o_ref):
        dn = (((1,), (0,)), ((), ()))
        h = x_ref[...]
        h = lax.dot_general(h, w1_ref[...], dn) + c1_ref[...]
        h = lax.dot_general(h, w2_ref[...], dn) + c2_ref[...]
        h = lax.dot_general(h, w3_ref[...], dn) + c3_ref[...]
        o_ref[...] = h

    wspec = pl.BlockSpec((4 * D, 4 * D), lambda i: (0, 0))
    bspec = pl.BlockSpec((1, 4 * D), lambda i: (0, 0))
    return pl.pallas_call(
        body,
        grid=(PR // BLK,),
        in_specs=[pl.BlockSpec((BLK, 4 * D), lambda i: (i, 0)),
                  wspec, bspec, wspec, bspec, wspec, bspec],
        out_specs=pl.BlockSpec((BLK, 4 * D), lambda i: (i, 0)),
        out_shape=jax.ShapeDtypeStruct((PR, 4 * D), jnp.float32),
    )(x128, w1, c1, w2, c2, w3, c3)


def kernel(indices, tables, W1, b1, W2, b2, W3, b3):
    offs = (jnp.arange(N_T, dtype=jnp.int32) * VOCAB)[:, None, None]
    idx3d = (indices.astype(jnp.int32) + offs).reshape(NW * CHUNKS, K, GW)
    flat_tab = _detile_tc(jnp.transpose(tables, (0, 2, 1))).reshape(N_T * VOCAB, D)
    pooled = _pooled_sc(idx3d, flat_tab)
    eye4 = jnp.eye(4, dtype=jnp.float32)
    out128 = _mlp_tc(pooled.reshape(PR, 4 * D),
                     jnp.kron(eye4, W1.T), jnp.tile(b1, 4).reshape(1, 4 * D),
                     jnp.kron(eye4, W2.T), jnp.tile(b2, 4).reshape(1, 4 * D),
                     jnp.kron(eye4, W3.T), jnp.tile(b3, 4).reshape(1, 4 * D))
    return out128.reshape(BAGS, D)


# speedup vs baseline: 6.5426x; 6.5426x over previous
import functools

import jax
import jax.numpy as jnp
from jax import lax
from jax.experimental import pallas as pl
from jax.experimental.pallas import tpu as pltpu
from jax.experimental.pallas import tpu_sc as plsc

N_T = 26
VOCAB = 100000
D = 32
BATCH = 4096
L = 20

BAGS = N_T * BATCH
NW = 32
BAGS_PER_W = BAGS // NW
G = 64
CHUNKS = BAGS_PER_W // G
IDX_PER_CHUNK = G * L
GW = 128
K = IDX_PER_CHUNK // GW
IDX_ROWS_PER_W = BAGS_PER_W * L // GW


def _pooled_sc(idx3d, flat_tab):
    mesh = plsc.VectorSubcoreMesh(core_axis_name="c", subcore_axis_name="s")

    @functools.partial(
        pl.kernel,
        out_type=jax.ShapeDtypeStruct((BAGS, D), jnp.float32),
        mesh=mesh,
        scratch_types=[
            pltpu.VMEM((K, GW), jnp.int32),
            pltpu.VMEM((IDX_PER_CHUNK, D), jnp.float32),
            pltpu.VMEM((G, D), jnp.float32),
            pltpu.SemaphoreType.DMA,
        ],
        compiler_params=pltpu.CompilerParams(use_tc_tiling_on_sc=False),
    )
    def k(idx_hbm, tab_hbm, out_hbm, idx_v, rows_v, out_v, sem):
        wid = lax.axis_index("s") * 2 + lax.axis_index("c")
        bag_base = wid * BAGS_PER_W

        @pl.loop(0, CHUNKS)
        def _(c):
            bag0 = bag_base + c * G
            pltpu.sync_copy(idx_hbm.at[wid * CHUNKS + c], idx_v)
            copies = []
            for j in range(K):
                copies.append(
                    pltpu.async_copy(
                        tab_hbm.at[idx_v.at[j]],
                        rows_v.at[pl.ds(j * GW, GW)],
                        sem,
                    )
                )
            for cp in copies:
                cp.wait()

            @pl.loop(0, G)
            def _(g):
                r0 = g * L
                a0 = rows_v[r0, pl.ds(0, 16)]
                a1 = rows_v[r0, pl.ds(16, 16)]
                for step in range(1, L):
                    a0 = a0 + rows_v[r0 + step, pl.ds(0, 16)]
                    a1 = a1 + rows_v[r0 + step, pl.ds(16, 16)]
                out_v[g, pl.ds(0, 16)] = a0
                out_v[g, pl.ds(16, 16)] = a1

            pltpu.sync_copy(out_v, out_hbm.at[pl.ds(bag0, G)])

    return k(idx3d, flat_tab)


VB = 12800
VB4 = VB // 4
NJ = -(-VOCAB // VB)


def _detile_tc(tab_t):
    def body(x_ref, i_ref, o_ref):
        x = x_ref[0]
        y = jnp.transpose(x, (1, 0))
        for c in range(4):
            o_ref[0, :, c * D:(c + 1) * D] = y[c * VB4:(c + 1) * VB4, :]

    return pl.pallas_call(
        body,
        grid=(N_T, NJ),
        in_specs=[pl.BlockSpec((1, D, VB), lambda t, j: (t, 0, j)),
                  pl.BlockSpec((D, D), lambda t, j: (0, 0))],
        out_specs=pl.BlockSpec((1, VB4, 4 * D), lambda t, j: (t, j, 0)),
        out_shape=jax.ShapeDtypeStruct((N_T, VOCAB // 4, 4 * D), jnp.float32),
        compiler_params=pltpu.CompilerParams(fuse_transposed_lhs_in_matmul=True),
    )(tab_t, jnp.eye(D, dtype=jnp.float32))


BLK = 2048
PR = BAGS // 4


def _mlp_tc(x128, w1, c1, w2, c2, w3, c3):
    def body(x_ref, w1_ref, c1_ref, w2_ref, c2_ref, w3_ref, c3_ref, o_ref):
        dn = (((1,), (0,)), ((), ()))
        h = x_ref[...]
        h = lax.dot_general(h, w1_ref[...], dn) + c1_ref[...]
        h = lax.dot_general(h, w2_ref[...], dn) + c2_ref[...]
        h = lax.dot_general(h, w3_ref[...], dn) + c3_ref[...]
        o_ref[...] = h

    wspec = pl.BlockSpec((4 * D, 4 * D), lambda i: (0, 0))
    bspec = pl.BlockSpec((1, 4 * D), lambda i: (0, 0))
    return pl.pallas_call(
        body,
        grid=(PR // BLK,),
        in_specs=[pl.BlockSpec((BLK, 4 * D), lambda i: (i, 0)),
                  wspec, bspec, wspec, bspec, wspec, bspec],
        out_specs=pl.BlockSpec((BLK, 4 * D), lambda i: (i, 0)),
        out_shape=jax.ShapeDtypeStruct((PR, 4 * D), jnp.float32),
    )(x128, w1, c1, w2, c2, w3, c3)


def kernel(indices, tables, W1, b1, W2, b2, W3, b3):
    v = indices.astype(jnp.int32)
    offs = (jnp.arange(N_T, dtype=jnp.int32) * VOCAB)[:, None, None]
    gidx = offs + (v // VB) * VB + 4 * (v % VB4) + (v % VB) // VB4
    idx3d = gidx.reshape(NW * CHUNKS, K, GW)
    flat_tab = _detile_tc(jnp.transpose(tables, (0, 2, 1))).reshape(N_T * VOCAB, D)
    pooled = _pooled_sc(idx3d, flat_tab)
    eye4 = jnp.eye(4, dtype=jnp.float32)
    out128 = _mlp_tc(pooled.reshape(PR, 4 * D),
                     jnp.kron(eye4, W1.T), jnp.tile(b1, 4).reshape(1, 4 * D),
                     jnp.kron(eye4, W2.T), jnp.tile(b2, 4).reshape(1, 4 * D),
                     jnp.kron(eye4, W3.T), jnp.tile(b3, 4).reshape(1, 4 * D))
    return out128.reshape(BAGS, D)


# speedup vs baseline: 6.5456x; 1.0005x over previous
import functools

import jax
import jax.numpy as jnp
from jax import lax
from jax.experimental import pallas as pl
from jax.experimental.pallas import tpu as pltpu
from jax.experimental.pallas import tpu_sc as plsc

N_T = 26
VOCAB = 100000
D = 32
BATCH = 4096
L = 20

BAGS = N_T * BATCH
NW = 32
BAGS_PER_W = BAGS // NW
G = 64
CHUNKS = BAGS_PER_W // G
IDX_PER_CHUNK = G * L
GW = 128
K = IDX_PER_CHUNK // GW
IDX_ROWS_PER_W = BAGS_PER_W * L // GW


def _pooled_sc(idx3d, flat_tab):
    mesh = plsc.VectorSubcoreMesh(core_axis_name="c", subcore_axis_name="s")

    @functools.partial(
        pl.kernel,
        out_type=jax.ShapeDtypeStruct((BAGS, D), jnp.float32),
        mesh=mesh,
        scratch_types=[
            pltpu.VMEM((K, GW), jnp.int32),
            pltpu.VMEM((IDX_PER_CHUNK, D), jnp.float32),
            pltpu.VMEM((G, D), jnp.float32),
            pltpu.SemaphoreType.DMA,
        ],
        compiler_params=pltpu.CompilerParams(use_tc_tiling_on_sc=False),
    )
    def k(idx_hbm, tab_hbm, out_hbm, idx_v, rows_v, out_v, sem):
        wid = lax.axis_index("s") * 2 + lax.axis_index("c")
        bag_base = wid * BAGS_PER_W

        @pl.loop(0, CHUNKS)
        def _(c):
            bag0 = bag_base + c * G
            pltpu.sync_copy(idx_hbm.at[wid * CHUNKS + c], idx_v)
            copies = []
            for j in range(K):
                copies.append(
                    pltpu.async_copy(
                        tab_hbm.at[idx_v.at[j]],
                        rows_v.at[pl.ds(j * GW, GW)],
                        sem,
                    )
                )
            for cp in copies:
                cp.wait()

            @pl.loop(0, G)
            def _(g):
                r0 = g * L
                a0 = rows_v[r0, pl.ds(0, 16)]
                a1 = rows_v[r0, pl.ds(16, 16)]
                for step in range(1, L):
                    a0 = a0 + rows_v[r0 + step, pl.ds(0, 16)]
                    a1 = a1 + rows_v[r0 + step, pl.ds(16, 16)]
                out_v[g, pl.ds(0, 16)] = a0
                out_v[g, pl.ds(16, 16)] = a1

            pltpu.sync_copy(out_v, out_hbm.at[pl.ds(bag0, G)])

    return k(idx3d, flat_tab)


VB = 12800
VB4 = VB // 4
NJ = -(-VOCAB // VB)
TV = NJ * VB


def _detile_tc(tab_t):
    def body(x_ref, i_ref, o_ref):
        x = x_ref[0]
        y = jnp.transpose(x, (1, 0))
        for c in range(4):
            o_ref[0, :, c * D:(c + 1) * D] = y[c * VB4:(c + 1) * VB4, :]

    return pl.pallas_call(
        body,
        grid=(N_T, NJ),
        in_specs=[pl.BlockSpec((1, D, VB), lambda t, j: (t, 0, j)),
                  pl.BlockSpec((D, D), lambda t, j: (0, 0))],
        out_specs=pl.BlockSpec((1, VB4, 4 * D), lambda t, j: (t, j, 0)),
        out_shape=jax.ShapeDtypeStruct((N_T, TV // 4, 4 * D), jnp.float32),
        compiler_params=pltpu.CompilerParams(fuse_transposed_lhs_in_matmul=True),
    )(tab_t, jnp.eye(D, dtype=jnp.float32))


BLK = 2048
PR = BAGS // 4


def _mlp_tc(x128, w1, c1, w2, c2, w3, c3):
    def body(x_ref, w1_ref, c1_ref, w2_ref, c2_ref, w3_ref, c3_ref, o_ref):
        dn = (((1,), (0,)), ((), ()))
        h = x_ref[...]
        h = lax.dot_general(h, w1_ref[...], dn) + c1_ref[...]
        h = lax.dot_general(h, w2_ref[...], dn) + c2_ref[...]
        h = lax.dot_general(h, w3_ref[...], dn) + c3_ref[...]
        o_ref[...] = h

    wspec = pl.BlockSpec((4 * D, 4 * D), lambda i: (0, 0))
    bspec = pl.BlockSpec((1, 4 * D), lambda i: (0, 0))
    return pl.pallas_call(
        body,
        grid=(PR // BLK,),
        in_specs=[pl.BlockSpec((BLK, 4 * D), lambda i: (i, 0)),
                  wspec, bspec, wspec, bspec, wspec, bspec],
        out_specs=pl.BlockSpec((BLK, 4 * D), lambda i: (i, 0)),
        out_shape=jax.ShapeDtypeStruct((PR, 4 * D), jnp.float32),
    )(x128, w1, c1, w2, c2, w3, c3)


def kernel(indices, tables, W1, b1, W2, b2, W3, b3):
    v = indices.astype(jnp.int32)
    offs = (jnp.arange(N_T, dtype=jnp.int32) * TV)[:, None, None]
    gidx = offs + (v // VB) * VB + 4 * (v % VB4) + (v % VB) // VB4
    idx3d = gidx.reshape(NW * CHUNKS, K, GW)
    flat_tab = _detile_tc(jnp.transpose(tables, (0, 2, 1))).reshape(N_T * TV, D)
    pooled = _pooled_sc(idx3d, flat_tab)
    eye4 = jnp.eye(4, dtype=jnp.float32)
    out128 = _mlp_tc(pooled.reshape(PR, 4 * D),
                     jnp.kron(eye4, W1.T), jnp.tile(b1, 4).reshape(1, 4 * D),
                     jnp.kron(eye4, W2.T), jnp.tile(b2, 4).reshape(1, 4 * D),
                     jnp.kron(eye4, W3.T), jnp.tile(b3, 4).reshape(1, 4 * D))
    return out128.reshape(BAGS, D)


# speedup vs baseline: 7.3069x; 1.1163x over previous
import functools

import jax
import jax.numpy as jnp
from jax import lax
from jax.experimental import pallas as pl
from jax.experimental.pallas import tpu as pltpu
from jax.experimental.pallas import tpu_sc as plsc

N_T = 26
VOCAB = 100000
D = 32
BATCH = 4096
L = 20

BAGS = N_T * BATCH
NW = 32
BAGS_PER_W = BAGS // NW
G = 64
CHUNKS = BAGS_PER_W // G
IDX_PER_CHUNK = G * L
GW = 128
K = IDX_PER_CHUNK // GW
IDX_ROWS_PER_W = BAGS_PER_W * L // GW


def _pooled_sc(idx3d, flat_tab):
    mesh = plsc.VectorSubcoreMesh(core_axis_name="c", subcore_axis_name="s")

    @functools.partial(
        pl.kernel,
        out_type=jax.ShapeDtypeStruct((BAGS, D), jnp.float32),
        mesh=mesh,
        scratch_types=[
            pltpu.VMEM((2, K, GW), jnp.int32),
            pltpu.VMEM((2, IDX_PER_CHUNK, D), jnp.float32),
            pltpu.VMEM((2, G, D), jnp.float32),
            pltpu.SemaphoreType.DMA,
            pltpu.SemaphoreType.DMA,
            pltpu.SemaphoreType.DMA,
        ],
        compiler_params=pltpu.CompilerParams(use_tc_tiling_on_sc=False),
    )
    def k(idx_hbm, tab_hbm, out_hbm, idx_v, rows_v, out_v, isem, gsem0, gsem1):
        wid = lax.axis_index("s") * 2 + lax.axis_index("c")
        bag_base = wid * BAGS_PER_W
        gsems = (gsem0, gsem1)

        def fetch_idx(c, b):
            pltpu.async_copy(idx_hbm.at[wid * CHUNKS + c], idx_v.at[b],
                             isem).wait()

        def fire_gathers(b):
            for j in range(K):
                pltpu.async_copy(tab_hbm.at[idx_v.at[b].at[j]],
                                 rows_v.at[b].at[pl.ds(j * GW, GW)], gsems[b])

        def wait_gathers(b):
            for j in range(K):
                pltpu.make_async_copy(tab_hbm.at[idx_v.at[b].at[j]],
                                      rows_v.at[b].at[pl.ds(j * GW, GW)],
                                      gsems[b]).wait()

        def pool_and_store(c, b):
            @pl.loop(0, G)
            def _(g):
                r0 = g * L
                a0 = rows_v[b, r0, pl.ds(0, 16)]
                a1 = rows_v[b, r0, pl.ds(16, 16)]
                for step in range(1, L):
                    a0 = a0 + rows_v[b, r0 + step, pl.ds(0, 16)]
                    a1 = a1 + rows_v[b, r0 + step, pl.ds(16, 16)]
                out_v[b, g, pl.ds(0, 16)] = a0
                out_v[b, g, pl.ds(16, 16)] = a1

            pltpu.sync_copy(out_v.at[b], out_hbm.at[pl.ds(bag_base + c * G, G)])

        fetch_idx(0, 0)
        fire_gathers(0)

        @pl.loop(0, CHUNKS, step=2)
        def _(c):
            fetch_idx(c + 1, 1)
            fire_gathers(1)
            wait_gathers(0)
            pool_and_store(c, 0)

            @pl.when(c + 2 < CHUNKS)
            def _():
                fetch_idx(c + 2, 0)
                fire_gathers(0)

            wait_gathers(1)
            pool_and_store(c + 1, 1)

    return k(idx3d, flat_tab)


VB = 12800
VB4 = VB // 4
NJ = -(-VOCAB // VB)
TV = NJ * VB


def _detile_tc(tab_t):
    def body(x_ref, i_ref, o_ref):
        x = x_ref[0]
        y = jnp.transpose(x, (1, 0))
        for c in range(4):
            o_ref[0, :, c * D:(c + 1) * D] = y[c * VB4:(c + 1) * VB4, :]

    return pl.pallas_call(
        body,
        grid=(N_T, NJ),
        in_specs=[pl.BlockSpec((1, D, VB), lambda t, j: (t, 0, j)),
                  pl.BlockSpec((D, D), lambda t, j: (0, 0))],
        out_specs=pl.BlockSpec((1, VB4, 4 * D), lambda t, j: (t, j, 0)),
        out_shape=jax.ShapeDtypeStruct((N_T, TV // 4, 4 * D), jnp.float32),
        compiler_params=pltpu.CompilerParams(fuse_transposed_lhs_in_matmul=True),
    )(tab_t, jnp.eye(D, dtype=jnp.float32))


BLK = 2048
PR = BAGS // 4


def _mlp_tc(x128, w1, c1, w2, c2, w3, c3):
    def body(x_ref, w1_ref, c1_ref, w2_ref, c2_ref, w3_ref, c3_ref, o_ref):
        dn = (((1,), (0,)), ((), ()))
        h = x_ref[...]
        h = lax.dot_general(h, w1_ref[...], dn) + c1_ref[...]
        h = lax.dot_general(h, w2_ref[...], dn) + c2_ref[...]
        h = lax.dot_general(h, w3_ref[...], dn) + c3_ref[...]
        o_ref[...] = h

    wspec = pl.BlockSpec((4 * D, 4 * D), lambda i: (0, 0))
    bspec = pl.BlockSpec((1, 4 * D), lambda i: (0, 0))
    return pl.pallas_call(
        body,
        grid=(PR // BLK,),
        in_specs=[pl.BlockSpec((BLK, 4 * D), lambda i: (i, 0)),
                  wspec, bspec, wspec, bspec, wspec, bspec],
        out_specs=pl.BlockSpec((BLK, 4 * D), lambda i: (i, 0)),
        out_shape=jax.ShapeDtypeStruct((PR, 4 * D), jnp.float32),
    )(x128, w1, c1, w2, c2, w3, c3)


def kernel(indices, tables, W1, b1, W2, b2, W3, b3):
    v = indices.astype(jnp.int32)
    offs = (jnp.arange(N_T, dtype=jnp.int32) * TV)[:, None, None]
    gidx = offs + (v // VB) * VB + 4 * (v % VB4) + (v % VB) // VB4
    idx3d = gidx.reshape(NW * CHUNKS, K, GW)
    flat_tab = _detile_tc(jnp.transpose(tables, (0, 2, 1))).reshape(N_T * TV, D)
    pooled = _pooled_sc(idx3d, flat_tab)
    eye4 = jnp.eye(4, dtype=jnp.float32)
    out128 = _mlp_tc(pooled.reshape(PR, 4 * D),
                     jnp.kron(eye4, W1.T), jnp.tile(b1, 4).reshape(1, 4 * D),
                     jnp.kron(eye4, W2.T), jnp.tile(b2, 4).reshape(1, 4 * D),
                     jnp.kron(eye4, W3.T), jnp.tile(b3, 4).reshape(1, 4 * D))
    return out128.reshape(BAGS, D)
